# Initial kernel scaffold; baseline (speedup 1.0000x reference)
import jax
import jax.numpy as jnp
from jax.experimental import pallas as pl


def kernel(tok_batch, tok_lens, emb_table):
    raise NotImplementedError("write your pallas kernel here")


# speedup vs baseline: 7.0167x; 7.0167x over previous
import jax
import jax.numpy as jnp
from jax.experimental import pallas as pl
from jax.experimental.pallas import tpu as pltpu


def _pool_kernel(tok_ref, len_ref, emb_hbm, out_ref, emb_vmem, sem):
    c = pl.program_id(0)
    j = pl.program_id(1)
    nj = pl.num_programs(1)
    tb = out_ref.shape[0]
    seq_len = tok_ref.shape[1]

    @pl.when(j == 0)
    def _():
        cp = pltpu.make_async_copy(emb_hbm, emb_vmem, sem)
        cp.start()
        cp.wait()

    base = (c * nj + j) * tb

    def row_body(r, carry):
        b = base + r
        acc = emb_vmem[tok_ref[b, 0]]
        for l in range(1, seq_len):
            acc = acc + emb_vmem[tok_ref[b, l]]
        out_ref[r] = acc / len_ref[b]
        return carry

    jax.lax.fori_loop(0, tb, row_body, 0)


def kernel(tok_batch, tok_lens, emb_table):
    B, L = tok_batch.shape
    V, D = emb_table.shape

    n_cores = 2
    tb = 128
    if B % (n_cores * tb) != 0:
        tb = B // n_cores
    tiles_per_core = B // (n_cores * tb)

    tok_i32 = tok_batch.astype(jnp.int32)
    lens_f32 = tok_lens.astype(jnp.float32)
    emb3 = emb_table.astype(jnp.float32).reshape(V, 1, D)

    grid_spec = pltpu.PrefetchScalarGridSpec(
        num_scalar_prefetch=2,
        grid=(n_cores, tiles_per_core),
        in_specs=[pl.BlockSpec(memory_space=pl.ANY)],
        out_specs=pl.BlockSpec(
            (tb, 1, D), lambda c, j, tok, lens: (c * tiles_per_core + j, 0, 0)
        ),
        scratch_shapes=[
            pltpu.VMEM((V, 1, D), jnp.float32),
            pltpu.SemaphoreType.DMA,
        ],
    )

    out = pl.pallas_call(
        _pool_kernel,
        out_shape=jax.ShapeDtypeStruct((B, 1, D), jnp.float32),
        grid_spec=grid_spec,
        compiler_params=pltpu.CompilerParams(
            dimension_semantics=("parallel", "arbitrary"),
            vmem_limit_bytes=44 << 20,
        ),
    )(tok_i32, lens_f32, emb3)
    return out.reshape(B, D)
